# baseline (speedup 1.0000x reference)
import jax
import jax.numpy as jnp
import numpy as np
from jax import lax
from jax.experimental import pallas as pl
from jax.experimental.pallas import tpu as pltpu
from jax.experimental.pallas import tpu_sc as plsc

_BATCH = 16384
_NSUB = 32
_K = 8192
_NC = 2
_NS = 16
_ROWS = _BATCH // _NS
_CHUNK = 256
_NCHUNK = (_ROWS * 16) // _CHUNK
_PIPE = 8
_TBL = _BATCH + 8


def _sc_hist(codes_hbm, table_hbm, out_hbm, data_v, idx_v, ones_v, zc_v,
             table_v, sums_v, hist_sh, stage_sem, tbl_sem, sem):
    c = lax.axis_index("c")
    s = lax.axis_index("s")
    stage = pltpu.async_copy(
        codes_hbm.at[pl.ds(s * _ROWS, _ROWS), pl.ds(c * 16, 16)], data_v,
        stage_sem)
    tstage = pltpu.async_copy(table_hbm, table_v, tbl_sem)

    def fill_ones(i, _):
        ones_v[pl.ds(i * 16, 16)] = jnp.full((16,), 1.0, jnp.float32)
        return 0
    lax.fori_loop(0, _CHUNK // 16, fill_ones, 0)

    def fill_zeros(i, _):
        zc_v[pl.ds(i * 16, 16)] = jnp.zeros((16,), jnp.float32)
        return 0
    lax.fori_loop(0, _K // 16, fill_zeros, 0)

    pltpu.sync_copy(zc_v, hist_sh.at[pl.ds(s * _K, _K)])
    stage.wait()
    plsc.subcore_barrier()

    lane_off = lax.iota(jnp.int32, 16) * _K

    def compute_and_fire(j):
        def vec(i, _):
            v = data_v[j * (_CHUNK // 16) + i]
            idx_v[j, pl.ds(i * 16, 16)] = v + lane_off
            return 0
        lax.fori_loop(0, _CHUNK // 16, vec, 0, unroll=True)
        pltpu.async_copy(ones_v, hist_sh.at[idx_v.at[j]], sem, add=True)

    def fire(j, _):
        compute_and_fire(j)
        return 0
    lax.fori_loop(0, _NCHUNK, fire, 0)

    def drain(j, _):
        pltpu.make_async_copy(ones_v, hist_sh.at[idx_v.at[j]], sem).wait()
        return 0
    lax.fori_loop(0, _NCHUNK, drain, 0)

    plsc.subcore_barrier()

    pltpu.sync_copy(hist_sh.at[pl.ds(s * _K, _K)], zc_v)
    tstage.wait()

    def gather(k, acc):
        cv = zc_v[pl.ds(k * 16, 16)]
        ci = cv.astype(jnp.int32)
        return acc + plsc.load_gather(table_v, [ci])
    acc = lax.fori_loop(0, _K // 16, gather,
                        jnp.zeros((16,), jnp.float32), unroll=2)
    sums_v[...] = acc
    pltpu.sync_copy(sums_v, out_hbm.at[pl.ds((c * 16 + s) * 16, 16)])


_hist_call = pl.kernel(
    _sc_hist,
    out_type=jax.ShapeDtypeStruct((_NSUB * 16,), jnp.float32),
    mesh=plsc.VectorSubcoreMesh(core_axis_name="c", subcore_axis_name="s",
                                num_cores=_NC, num_subcores=_NS),
    scratch_types=[
        pltpu.VMEM((_ROWS, 16), jnp.int32),
        pltpu.VMEM((_NCHUNK, _CHUNK), jnp.int32),
        pltpu.VMEM((_CHUNK,), jnp.float32),
        pltpu.VMEM((_K,), jnp.float32),
        pltpu.VMEM((_TBL,), jnp.float32),
        pltpu.VMEM((16,), jnp.float32),
        pltpu.VMEM_SHARED((_NS * _K,), jnp.float32),
        pltpu.SemaphoreType.DMA,
        pltpu.SemaphoreType.DMA,
        pltpu.SemaphoreType.DMA,
    ],
    compiler_params=pltpu.CompilerParams(use_tc_tiling_on_sc=False,
                                         needs_layout_passes=False),
)


def _f_table():
    n = np.arange(_TBL, dtype=np.float64)
    q = n / _BATCH + 1e-8
    z = 1.0 + _K * 1e-8
    p = q / z
    return jnp.asarray(p * np.log(p), dtype=jnp.float32)


def _tc_loss(sums_ref, out_ref):
    x = sums_ref[...]
    neg_ent = jnp.sum(x, axis=1, keepdims=True)
    target = jnp.log(jnp.float32(_K))
    d = target + neg_ent
    out_ref[0, 0] = jnp.sum(d * d) * (1.0 / _NSUB)


def kernel(codes):
    sums = _hist_call(codes, _f_table()).reshape(_NSUB, 16)
    loss = pl.pallas_call(
        _tc_loss,
        out_shape=jax.ShapeDtypeStruct((1, 1), jnp.float32),
        out_specs=pl.BlockSpec(memory_space=pltpu.SMEM),
    )(sums)
    return loss[0, 0]


# speedup vs baseline: 1.0416x; 1.0416x over previous
import jax
import jax.numpy as jnp
import numpy as np
from jax import lax
from jax.experimental import pallas as pl
from jax.experimental.pallas import tpu as pltpu
from jax.experimental.pallas import tpu_sc as plsc

_BATCH = 16384
_NSUB = 32
_K = 8192
_NC = 2
_NS = 16
_ROWS = _BATCH // _NS
_CHUNK = 128
_NCHUNK = (_ROWS * 16) // _CHUNK
_PIPE = 8
_TBL = _BATCH + 8


def _sc_hist(codes_hbm, table_hbm, out_hbm, data_v, idx_v, ones_v, zc_v,
             table_v, sums_v, hist_sh, stage_sem, tbl_sem, sem):
    c = lax.axis_index("c")
    s = lax.axis_index("s")
    stage = pltpu.async_copy(
        codes_hbm.at[pl.ds(s * _ROWS, _ROWS), pl.ds(c * 16, 16)], data_v,
        stage_sem)
    tstage = pltpu.async_copy(table_hbm, table_v, tbl_sem)

    def fill_ones(i, _):
        ones_v[pl.ds(i * 16, 16)] = jnp.full((16,), 1.0, jnp.float32)
        return 0
    lax.fori_loop(0, _CHUNK // 16, fill_ones, 0)

    def fill_zeros(i, _):
        zc_v[pl.ds(i * 16, 16)] = jnp.zeros((16,), jnp.float32)
        return 0
    lax.fori_loop(0, _K // 16, fill_zeros, 0)

    pltpu.sync_copy(zc_v, hist_sh.at[pl.ds(s * _K, _K)])
    stage.wait()
    plsc.subcore_barrier()

    lane_off = lax.iota(jnp.int32, 16) * _K

    def compute_and_fire(j):
        def vec(i, _):
            v = data_v[j * (_CHUNK // 16) + i]
            idx_v[j, pl.ds(i * 16, 16)] = v + lane_off
            return 0
        lax.fori_loop(0, _CHUNK // 16, vec, 0, unroll=True)
        pltpu.async_copy(ones_v, hist_sh.at[idx_v.at[j]], sem, add=True)

    def fire(j, _):
        compute_and_fire(j)
        return 0
    lax.fori_loop(0, _NCHUNK, fire, 0)

    def drain(j, _):
        pltpu.make_async_copy(ones_v, hist_sh.at[idx_v.at[j]], sem).wait()
        return 0
    lax.fori_loop(0, _NCHUNK, drain, 0)

    plsc.subcore_barrier()

    pltpu.sync_copy(hist_sh.at[pl.ds(s * _K, _K)], zc_v)
    tstage.wait()

    def gather(k, acc):
        cv = zc_v[pl.ds(k * 16, 16)]
        ci = cv.astype(jnp.int32)
        return acc + plsc.load_gather(table_v, [ci])
    acc = lax.fori_loop(0, _K // 16, gather,
                        jnp.zeros((16,), jnp.float32), unroll=4)
    sums_v[...] = acc
    pltpu.sync_copy(sums_v, out_hbm.at[pl.ds((c * 16 + s) * 16, 16)])


_hist_call = pl.kernel(
    _sc_hist,
    out_type=jax.ShapeDtypeStruct((_NSUB * 16,), jnp.float32),
    mesh=plsc.VectorSubcoreMesh(core_axis_name="c", subcore_axis_name="s",
                                num_cores=_NC, num_subcores=_NS),
    scratch_types=[
        pltpu.VMEM((_ROWS, 16), jnp.int32),
        pltpu.VMEM((_NCHUNK, _CHUNK), jnp.int32),
        pltpu.VMEM((_CHUNK,), jnp.float32),
        pltpu.VMEM((_K,), jnp.float32),
        pltpu.VMEM((_TBL,), jnp.float32),
        pltpu.VMEM((16,), jnp.float32),
        pltpu.VMEM_SHARED((_NS * _K,), jnp.float32),
        pltpu.SemaphoreType.DMA,
        pltpu.SemaphoreType.DMA,
        pltpu.SemaphoreType.DMA,
    ],
    compiler_params=pltpu.CompilerParams(use_tc_tiling_on_sc=False,
                                         needs_layout_passes=False),
)


def _f_table():
    n = np.arange(_TBL, dtype=np.float64)
    q = n / _BATCH + 1e-8
    z = 1.0 + _K * 1e-8
    p = q / z
    return jnp.asarray(p * np.log(p), dtype=jnp.float32)


def _tc_loss(sums_ref, out_ref):
    x = sums_ref[...].reshape(1, _NSUB * 16)
    i2 = lax.broadcasted_iota(jnp.int32, (_NSUB * 16, _NSUB), 0) // 16
    j2 = lax.broadcasted_iota(jnp.int32, (_NSUB * 16, _NSUB), 1)
    g = (i2 == j2).astype(jnp.float32)
    neg_ent = jnp.dot(x, g, preferred_element_type=jnp.float32)
    target = jnp.log(jnp.float32(_K))
    d = target + neg_ent
    out_ref[0, 0] = jnp.sum(d * d) * (1.0 / _NSUB)


def kernel(codes):
    sums = _hist_call(codes, _f_table())
    loss = pl.pallas_call(
        _tc_loss,
        out_shape=jax.ShapeDtypeStruct((1, 1), jnp.float32),
        out_specs=pl.BlockSpec(memory_space=pltpu.SMEM),
    )(sums)
    return loss[0, 0]


# speedup vs baseline: 1.0439x; 1.0021x over previous
import jax
import jax.numpy as jnp
import numpy as np
from jax import lax
from jax.experimental import pallas as pl
from jax.experimental.pallas import tpu as pltpu
from jax.experimental.pallas import tpu_sc as plsc

_BATCH = 16384
_NSUB = 32
_K = 8192
_NC = 2
_NS = 16
_ROWS = _BATCH // _NS
_CHUNK = 128
_NCHUNK = (_ROWS * 16) // _CHUNK
_PIPE = 8
_TBL = _BATCH + 8


def _sc_hist(codes_hbm, table_hbm, out_hbm, data_v, idx_v, ones_v, zc_v,
             table_v, sums_v, hist_sh, stage_sem, tbl_sem, sem):
    c = lax.axis_index("c")
    s = lax.axis_index("s")
    stage = pltpu.async_copy(
        codes_hbm.at[pl.ds(s * _ROWS, _ROWS), pl.ds(c * 16, 16)], data_v,
        stage_sem)
    tstage = pltpu.async_copy(table_hbm, table_v, tbl_sem)

    def fill_ones(i, _):
        ones_v[pl.ds(i * 16, 16)] = jnp.full((16,), 1.0, jnp.float32)
        return 0
    lax.fori_loop(0, _CHUNK // 16, fill_ones, 0)

    def fill_zeros(i, _):
        zc_v[pl.ds(i * 16, 16)] = jnp.zeros((16,), jnp.float32)
        return 0
    lax.fori_loop(0, _K // 16, fill_zeros, 0)

    pltpu.sync_copy(zc_v, hist_sh.at[pl.ds(s * _K, _K)])
    stage.wait()
    plsc.subcore_barrier()

    lane_off = lax.iota(jnp.int32, 16) * _K

    def compute_and_fire(j):
        def vec(i, _):
            v = data_v[j * (_CHUNK // 16) + i]
            idx_v[j, pl.ds(i * 16, 16)] = v + lane_off
            return 0
        lax.fori_loop(0, _CHUNK // 16, vec, 0, unroll=True)
        pltpu.async_copy(ones_v, hist_sh.at[idx_v.at[j]], sem, add=True)

    def fire(j, _):
        compute_and_fire(j)
        return 0
    lax.fori_loop(0, _NCHUNK, fire, 0)

    def drain(j, _):
        pltpu.make_async_copy(ones_v, hist_sh.at[idx_v.at[j]], sem).wait()
        return 0
    lax.fori_loop(0, _NCHUNK, drain, 0, unroll=4)

    plsc.subcore_barrier()

    pltpu.sync_copy(hist_sh.at[pl.ds(s * _K, _K)], zc_v)
    tstage.wait()

    def gather(k, acc):
        cv = zc_v[pl.ds(k * 16, 16)]
        ci = cv.astype(jnp.int32)
        return acc + plsc.load_gather(table_v, [ci])
    acc = lax.fori_loop(0, _K // 16, gather,
                        jnp.zeros((16,), jnp.float32), unroll=4)
    sums_v[...] = jnp.broadcast_to(jnp.sum(acc), (16,))
    pltpu.sync_copy(sums_v, out_hbm.at[pl.ds((c * 16 + s) * 16, 16)])


_hist_call = pl.kernel(
    _sc_hist,
    out_type=jax.ShapeDtypeStruct((_NSUB * 16,), jnp.float32),
    mesh=plsc.VectorSubcoreMesh(core_axis_name="c", subcore_axis_name="s",
                                num_cores=_NC, num_subcores=_NS),
    scratch_types=[
        pltpu.VMEM((_ROWS, 16), jnp.int32),
        pltpu.VMEM((_NCHUNK, _CHUNK), jnp.int32),
        pltpu.VMEM((_CHUNK,), jnp.float32),
        pltpu.VMEM((_K,), jnp.float32),
        pltpu.VMEM((_TBL,), jnp.float32),
        pltpu.VMEM((16,), jnp.float32),
        pltpu.VMEM_SHARED((_NS * _K,), jnp.float32),
        pltpu.SemaphoreType.DMA,
        pltpu.SemaphoreType.DMA,
        pltpu.SemaphoreType.DMA,
    ],
    compiler_params=pltpu.CompilerParams(use_tc_tiling_on_sc=False,
                                         needs_layout_passes=False),
)


def _f_table():
    n = np.arange(_TBL, dtype=np.float64)
    q = n / _BATCH + 1e-8
    z = 1.0 + _K * 1e-8
    p = q / z
    return jnp.asarray(p * np.log(p), dtype=jnp.float32)


def _tc_loss(sums_ref, out_ref):
    x = sums_ref[...]
    target = jnp.log(jnp.float32(_K))
    d = target + x
    out_ref[0, 0] = jnp.sum(d * d) * (1.0 / (_NSUB * 16))


def kernel(codes):
    sums = _hist_call(codes, _f_table())
    loss = pl.pallas_call(
        _tc_loss,
        out_shape=jax.ShapeDtypeStruct((1, 1), jnp.float32),
        out_specs=pl.BlockSpec(memory_space=pltpu.SMEM),
    )(sums)
    return loss[0, 0]


# speedup vs baseline: 1.0491x; 1.0051x over previous
import jax
import jax.numpy as jnp
import numpy as np
from jax import lax
from jax.experimental import pallas as pl
from jax.experimental.pallas import tpu as pltpu
from jax.experimental.pallas import tpu_sc as plsc

_BATCH = 16384
_NSUB = 32
_K = 8192
_NC = 2
_NS = 16
_ROWS = _BATCH // _NS
_CHUNK = 128
_NCHUNK = (_ROWS * 16) // _CHUNK
_TBL = _BATCH + 8


def _sc_hist(codes_hbm, table_hbm, out_hbm, data_v, idx_v, ones_v, zc_v,
             table_v, sums_v, hist_sh, stage_sem, tbl_sem, sem):
    c = lax.axis_index("c")
    s = lax.axis_index("s")
    stage = pltpu.async_copy(
        codes_hbm.at[pl.ds(s * _ROWS, _ROWS), pl.ds(c * 16, 16)], data_v,
        stage_sem)
    tstage = pltpu.async_copy(table_hbm, table_v, tbl_sem)

    def fill_ones(i, _):
        ones_v[pl.ds(i * 16, 16)] = jnp.full((16,), 1.0, jnp.float32)
        return 0
    lax.fori_loop(0, _CHUNK // 16, fill_ones, 0)

    def fill_zeros(i, _):
        zc_v[pl.ds(i * 16, 16)] = jnp.zeros((16,), jnp.float32)
        return 0
    lax.fori_loop(0, _K // 16, fill_zeros, 0)

    pltpu.sync_copy(zc_v, hist_sh.at[pl.ds(s * _K, _K)])
    stage.wait()
    plsc.subcore_barrier()

    lane_off = lax.iota(jnp.int32, 16) * _K

    def compute_and_fire(j):
        def vec(i, _):
            v = data_v[j * (_CHUNK // 16) + i]
            idx_v[j, pl.ds(i * 16, 16)] = v + lane_off
            return 0
        lax.fori_loop(0, _CHUNK // 16, vec, 0, unroll=True)
        pltpu.async_copy(ones_v, hist_sh.at[idx_v.at[j]], sem, add=True)

    def fire(j, _):
        compute_and_fire(j)
        return 0
    lax.fori_loop(0, _NCHUNK, fire, 0)

    def drain(j, _):
        pltpu.make_async_copy(ones_v, hist_sh.at[idx_v.at[j]], sem).wait()
        return 0
    lax.fori_loop(0, _NCHUNK, drain, 0, unroll=4)

    plsc.subcore_barrier()

    pltpu.sync_copy(hist_sh.at[pl.ds(s * _K, _K)], zc_v)
    tstage.wait()

    def gather(k, acc):
        cv = zc_v[pl.ds(k * 16, 16)]
        ci = cv.astype(jnp.int32)
        return acc + plsc.load_gather(table_v, [ci])
    acc = lax.fori_loop(0, _K // 16, gather,
                        jnp.zeros((16,), jnp.float32), unroll=4)
    sums_v[...] = jnp.broadcast_to(jnp.sum(acc), (16,))
    pltpu.sync_copy(sums_v, out_hbm.at[pl.ds((c * 16 + s) * 16, 16)])


_hist_call = pl.kernel(
    _sc_hist,
    out_type=jax.ShapeDtypeStruct((_NSUB * 16,), jnp.float32),
    mesh=plsc.VectorSubcoreMesh(core_axis_name="c", subcore_axis_name="s",
                                num_cores=_NC, num_subcores=_NS),
    scratch_types=[
        pltpu.VMEM((_ROWS, 16), jnp.int32),
        pltpu.VMEM((_NCHUNK, _CHUNK), jnp.int32),
        pltpu.VMEM((_CHUNK,), jnp.float32),
        pltpu.VMEM((_K,), jnp.float32),
        pltpu.VMEM((_TBL,), jnp.float32),
        pltpu.VMEM((16,), jnp.float32),
        pltpu.VMEM_SHARED((_NS * _K,), jnp.float32),
        pltpu.SemaphoreType.DMA,
        pltpu.SemaphoreType.DMA,
        pltpu.SemaphoreType.DMA,
    ],
    compiler_params=pltpu.CompilerParams(use_tc_tiling_on_sc=False,
                                         needs_layout_passes=False),
)


def _f_table():
    n = np.arange(_TBL, dtype=np.float64)
    q = n / _BATCH + 1e-8
    z = 1.0 + _K * 1e-8
    p = q / z
    return jnp.asarray(p * np.log(p), dtype=jnp.float32)


def _tc_loss(sums_ref, out_ref):
    x = sums_ref[...]
    target = jnp.log(jnp.float32(_K))
    d = target + x
    out_ref[0, 0] = jnp.sum(d * d) * (1.0 / (_NSUB * 16))


def kernel(codes):
    sums = _hist_call(codes, _f_table())
    loss = pl.pallas_call(
        _tc_loss,
        out_shape=jax.ShapeDtypeStruct((1, 1), jnp.float32),
        out_specs=pl.BlockSpec(memory_space=pltpu.SMEM),
    )(sums)
    return loss[0, 0]
